# baseline (speedup 1.0000x reference)
import functools

import jax
import jax.numpy as jnp
from jax import lax
from jax.experimental import pallas as pl
from jax.experimental.pallas import tpu as pltpu
from jax.experimental.pallas import tpu_sc as plsc

EMB = 64
PAD = 0
NEG = -99999999.0

NC, NS = 2, 16
NW = NC * NS

CH = 128
NBUF = 4


def _sc_gather_rows(idx2d, table2, total_rows, big, small):
    nrow = total_rows // CH
    assert 3 * big + small == nrow // 8 and big % 8 == 0 and small % 8 == 0
    mesh = plsc.VectorSubcoreMesh(
        core_axis_name="c", subcore_axis_name="s",
        num_cores=NC, num_subcores=NS)

    @functools.partial(
        pl.kernel,
        out_type=jax.ShapeDtypeStruct((total_rows, 2 * EMB), jnp.float32),
        mesh=mesh,
        scratch_types=[
            pltpu.VMEM((big, CH), jnp.int32),
            pltpu.VMEM((NBUF, CH, 2 * EMB), jnp.float32),
            [pltpu.SemaphoreType.DMA] * NBUF,
            [pltpu.SemaphoreType.DMA] * NBUF,
        ],
    )
    def k(idx_hbm, table_hbm, out_hbm, idx_v, rows_v, gsems, osems):
        wid = lax.axis_index("s") * NC + lax.axis_index("c")
        m = wid % 4
        is_big = m < 3
        base_row = (wid // 4) * (3 * big + small) + m * big
        nch = jnp.where(is_big, big, small)
        out0 = base_row * CH

        pltpu.sync_copy(idx_hbm.at[pl.ds(base_row, small)],
                        idx_v.at[pl.ds(0, small)])

        @pl.when(is_big)
        def _():
            pltpu.sync_copy(idx_hbm.at[pl.ds(base_row + small, big - small)],
                            idx_v.at[pl.ds(small, big - small)])

        def g_copy(c, q):
            return pltpu.make_async_copy(
                table_hbm.at[idx_v.at[c]], rows_v.at[q], gsems[q])

        def o_copy(c, q):
            return pltpu.make_async_copy(
                rows_v.at[q], out_hbm.at[pl.ds(out0 + c * CH, CH)], osems[q])

        for q in range(NBUF):
            g_copy(q, q).start()

        def step(i, carry):
            c0 = i * NBUF
            for q in range(NBUF):
                g_copy(c0 + q, q).wait()
                o_copy(c0 + q, q).start()

                @pl.when(c0 + NBUF + q < nch)
                def _(q=q):
                    o_copy(c0 + q, q).wait()
                    g_copy(c0 + NBUF + q, q).start()
            return carry

        lax.fori_loop(0, nch // NBUF, step, 0)
        for q in range(NBUF):
            o_copy(0, q).wait()

    return k(idx2d, table2)


BLK = 512


def _tc_transpose_pad(table_t, vocab):
    C = 4096
    grid = (pl.cdiv(vocab, C),)

    def body(t_ref, out_ref):
        x = t_ref[...]
        xt = jnp.swapaxes(x, 0, 1)
        out_ref[...] = jnp.concatenate(
            [xt, jnp.zeros((C, EMB), jnp.float32)], axis=1)

    return pl.pallas_call(
        body,
        grid=grid,
        in_specs=[pl.BlockSpec((EMB, C), lambda i: (0, i))],
        out_specs=pl.BlockSpec((C, 2 * EMB), lambda i: (i, 0)),
        out_shape=jax.ShapeDtypeStruct((vocab, 2 * EMB), jnp.float32),
    )(table_t)


def _tc_dense(gpairs, tok, pos, pos_weight, bsz_num):
    S = tok.shape[1]
    grid = (bsz_num // BLK,)
    pwt = jnp.pad(pos_weight, ((0, 1), (0, 0))).T

    def body(g_ref, tok_ref, pos_ref, pwt_ref, out_ref):
        ep = g_ref[...]
        t3 = tok_ref[...].reshape(BLK, S, 1)
        e = ep[:, :EMB].reshape(BLK, S, EMB)
        e2 = e.reshape(BLK * S, EMB)
        s7 = jnp.dot(e2, pwt_ref[...], preferred_element_type=jnp.float32,
                     precision=lax.Precision.HIGHEST)
        s73 = s7.reshape(BLK, S, 8)
        p3 = pos_ref[...].reshape(BLK, S, 1)
        i3 = lax.broadcasted_iota(jnp.int32, (BLK, S, 8), 2)
        sc = jnp.sum(jnp.where(i3 == p3, s73, 0.0), axis=2, keepdims=True)
        sc = jnp.where(t3 == PAD, NEG, sc)
        m = jnp.max(sc, axis=1, keepdims=True)
        w = jnp.exp(sc - m)
        w = w / jnp.sum(w, axis=1, keepdims=True)
        out_ref[...] = jnp.sum(w * e, axis=1)

    return pl.pallas_call(
        body,
        grid=grid,
        in_specs=[
            pl.BlockSpec((BLK * S, 2 * EMB), lambda i: (i, 0)),
            pl.BlockSpec((BLK, S), lambda i: (i, 0)),
            pl.BlockSpec((BLK, S), lambda i: (i, 0)),
            pl.BlockSpec((EMB, 8), lambda i: (0, 0)),
        ],
        out_specs=pl.BlockSpec((BLK, EMB), lambda i: (i, 0)),
        out_shape=jax.ShapeDtypeStruct((bsz_num, EMB), jnp.float32),
    )(gpairs, tok, pos, pwt)


def kernel(chld_prt_tokens, types, positions, embed_weight, pos_weight):
    bsz, num, seq_len = chld_prt_tokens.shape
    bn = bsz * num
    total = bn * seq_len
    tok2d = chld_prt_tokens.reshape(bn, seq_len)
    pos2d = positions.reshape(bn, seq_len)
    SP = 24
    tok24 = jnp.pad(tok2d, ((0, 0), (0, SP - seq_len)))
    pos24 = jnp.pad(pos2d, ((0, 0), (0, SP - seq_len)))
    total24 = bn * SP
    idx2d = tok24.reshape(total24 // 128, 128)
    table2 = _tc_transpose_pad(embed_weight.T, embed_weight.shape[0])

    hr = total24 // 2
    hn = bn // 2
    hi2 = idx2d.shape[0] // 2
    g1 = _sc_gather_rows(idx2d[:hi2], table2, hr, 64, 48)
    g2 = _sc_gather_rows(idx2d[hi2:], table2, hr, 64, 48)
    r1 = _tc_dense(g1, tok24[:hn], pos24[:hn], pos_weight, hn)
    r2 = _tc_dense(g2, tok24[hn:], pos24[hn:], pos_weight, hn)
    res = jnp.concatenate([r1, r2], axis=0)
    return res.reshape(bsz, num, EMB)


# speedup vs baseline: 4.8416x; 4.8416x over previous
import functools

import jax
import jax.numpy as jnp
from jax import lax
from jax.experimental import pallas as pl
from jax.experimental.pallas import tpu as pltpu
from jax.experimental.pallas import tpu_sc as plsc

EMB = 64
PAD = 0
NEG = -99999999.0

NC, NS = 2, 16
NW = NC * NS

CH = 128
NBUF = 4


def _sc_gather_rows(idx2d, table2, total_rows, big, small):
    nrow = total_rows // CH
    assert 3 * big + small == nrow // 8 and big % 8 == 0 and small % 8 == 0
    mesh = plsc.VectorSubcoreMesh(
        core_axis_name="c", subcore_axis_name="s",
        num_cores=NC, num_subcores=NS)

    @functools.partial(
        pl.kernel,
        out_type=jax.ShapeDtypeStruct((total_rows, 2 * EMB), jnp.float32),
        mesh=mesh,
        scratch_types=[
            pltpu.VMEM((big, CH), jnp.int32),
            pltpu.VMEM((NBUF, CH, 2 * EMB), jnp.float32),
            [pltpu.SemaphoreType.DMA] * NBUF,
            [pltpu.SemaphoreType.DMA] * NBUF,
        ],
    )
    def k(idx_hbm, table_hbm, out_hbm, idx_v, rows_v, gsems, osems):
        wid = lax.axis_index("s") * NC + lax.axis_index("c")
        m = wid % 4
        is_big = m < 3
        base_row = (wid // 4) * (3 * big + small) + m * big
        nch = jnp.where(is_big, big, small)
        out0 = base_row * CH

        pltpu.sync_copy(idx_hbm.at[pl.ds(base_row, small)],
                        idx_v.at[pl.ds(0, small)])

        @pl.when(is_big)
        def _():
            pltpu.sync_copy(idx_hbm.at[pl.ds(base_row + small, big - small)],
                            idx_v.at[pl.ds(small, big - small)])

        def g_copy(c, q):
            return pltpu.make_async_copy(
                table_hbm.at[idx_v.at[c]], rows_v.at[q], gsems[q])

        def o_copy(c, q):
            return pltpu.make_async_copy(
                rows_v.at[q], out_hbm.at[pl.ds(out0 + c * CH, CH)], osems[q])

        for q in range(NBUF):
            g_copy(q, q).start()

        def step(i, carry):
            c0 = i * NBUF
            for q in range(NBUF):
                g_copy(c0 + q, q).wait()
                o_copy(c0 + q, q).start()

                @pl.when(c0 + NBUF + q < nch)
                def _(q=q):
                    o_copy(c0 + q, q).wait()
                    g_copy(c0 + NBUF + q, q).start()
            return carry

        lax.fori_loop(0, nch // NBUF, step, 0)
        for q in range(NBUF):
            o_copy(0, q).wait()

    return k(idx2d, table2)


BLK = 512


def _tc_transpose_pad(table_t, vocab):
    C = 4096
    grid = (pl.cdiv(vocab, C),)

    def body(t_ref, out_ref):
        x = t_ref[...]
        xt = jnp.swapaxes(x, 0, 1)
        out_ref[...] = jnp.concatenate(
            [xt, jnp.zeros((C, EMB), jnp.float32)], axis=1)

    return pl.pallas_call(
        body,
        grid=grid,
        in_specs=[pl.BlockSpec((EMB, C), lambda i: (0, i))],
        out_specs=pl.BlockSpec((C, 2 * EMB), lambda i: (i, 0)),
        out_shape=jax.ShapeDtypeStruct((vocab, 2 * EMB), jnp.float32),
    )(table_t)


def _tc_dense(gpairs, tok, pos, pos_weight, bsz_num):
    S = tok.shape[1]
    grid = (bsz_num // BLK,)
    pwt = jnp.pad(pos_weight, ((0, 1), (0, 0))).T

    def body(g_ref, tok_ref, pos_ref, pwt_ref, out_ref):
        ep = g_ref[...]
        t3 = tok_ref[...].reshape(BLK, S, 1)
        e = ep[:, :EMB].reshape(BLK, S, EMB)
        e2 = e.reshape(BLK * S, EMB)
        s7 = jnp.dot(e2, pwt_ref[...], preferred_element_type=jnp.float32,
                     precision=lax.Precision.HIGHEST)
        s73 = s7.reshape(BLK, S, 8)
        p3 = pos_ref[...].reshape(BLK, S, 1)
        i3 = lax.broadcasted_iota(jnp.int32, (BLK, S, 8), 2)
        sc = jnp.sum(jnp.where(i3 == p3, s73, 0.0), axis=2, keepdims=True)
        sc = jnp.where(t3 == PAD, NEG, sc)
        m = jnp.max(sc, axis=1, keepdims=True)
        w = jnp.exp(sc - m)
        w = w / jnp.sum(w, axis=1, keepdims=True)
        out_ref[...] = jnp.sum(w * e, axis=1)

    return pl.pallas_call(
        body,
        grid=grid,
        in_specs=[
            pl.BlockSpec((BLK * S, 2 * EMB), lambda i: (i, 0)),
            pl.BlockSpec((BLK, S), lambda i: (i, 0)),
            pl.BlockSpec((BLK, S), lambda i: (i, 0)),
            pl.BlockSpec((EMB, 8), lambda i: (0, 0)),
        ],
        out_specs=pl.BlockSpec((BLK, EMB), lambda i: (i, 0)),
        out_shape=jax.ShapeDtypeStruct((bsz_num, EMB), jnp.float32),
    )(gpairs, tok, pos, pwt)


def kernel(chld_prt_tokens, types, positions, embed_weight, pos_weight):
    bsz, num, seq_len = chld_prt_tokens.shape
    bn = bsz * num
    total = bn * seq_len
    tok2d = chld_prt_tokens.reshape(bn, seq_len)
    pos2d = positions.reshape(bn, seq_len)
    SP = 24
    tok24 = jnp.pad(tok2d, ((0, 0), (0, SP - seq_len)))
    pos24 = jnp.pad(pos2d, ((0, 0), (0, SP - seq_len)))
    idx24 = jnp.concatenate([tok2d, tok2d[:, :SP - seq_len]], axis=1)
    total24 = bn * SP
    idx2d = idx24.reshape(total24 // 128, 128)
    table2 = _tc_transpose_pad(embed_weight.T, embed_weight.shape[0])

    hr = total24 // 2
    hn = bn // 2
    hi2 = idx2d.shape[0] // 2
    g1 = _sc_gather_rows(idx2d[:hi2], table2, hr, 64, 48)
    g2 = _sc_gather_rows(idx2d[hi2:], table2, hr, 64, 48)
    r1 = _tc_dense(g1, tok24[:hn], pos24[:hn], pos_weight, hn)
    r2 = _tc_dense(g2, tok24[hn:], pos24[hn:], pos_weight, hn)
    res = jnp.concatenate([r1, r2], axis=0)
    return res.reshape(bsz, num, EMB)
